# baseline (speedup 1.0000x reference)
import functools

import jax
import jax.numpy as jnp
from jax import lax
from jax.experimental import pallas as pl
from jax.experimental.pallas import tpu as pltpu
from jax.experimental.pallas import tpu_sc as plsc

N_NODES = 10000
N_EDGES = 320000
UNITS = 128
EPS = 1e-08
BN_EPS = 1e-05

NC = 2
NS = 16

EPT = N_EDGES // (NC * NS)
K = 80
NBLK = EPT // K

STRIPE_A = 624
STRIPE_LAST = N_NODES - (NS - 1) * STRIPE_A
DSTRIPE_A = 640
DSTRIPE_LAST = N_NODES - (NS - 1) * DSTRIPE_A


def _y2_body(x_ref, w2_ref, y_ref):
    y_ref[...] = jnp.dot(x_ref[...], w2_ref[...].T,
                         preferred_element_type=jnp.float32)


def _y2_call(x, W2):
    return pl.pallas_call(
        _y2_body,
        out_shape=jax.ShapeDtypeStruct((N_NODES, UNITS), jnp.float32),
    )(x, W2)


_ES_BLK = 4000


def _es_body(e_ref, o_ref):
    o_ref[...] = jax.nn.sigmoid(e_ref[...])


def _es_call(e):
    return pl.pallas_call(
        _es_body,
        grid=(N_EDGES // _ES_BLK,),
        in_specs=[pl.BlockSpec((_ES_BLK, UNITS), lambda i: (i, 0))],
        out_specs=pl.BlockSpec((_ES_BLK, UNITS), lambda i: (i, 0)),
        out_shape=jax.ShapeDtypeStruct((N_EDGES, UNITS), jnp.float32),
    )(e)


def _edge_body(e_h, src_h, dst_h, y2_h,
               p_h, u_h, deg0_h, deg1_h,
               eb0, eb1, yb0, yb1,
               ix0, ix1, ix2, ix3,
               id0, id1, id2, id3,
               onesb, zbuf,
               acc, dacc,
               ls0, ls1, is0, is1, gs0, gs1, ss0, ss1):
    c = lax.axis_index("c")
    s = lax.axis_index("s")

    E4 = [eb0, eb1, yb0, yb1]
    EU = [eb0, eb1]
    YU = [yb0, yb1]
    IXS = [ix0, ix1, ix2, ix3]
    IXD = [id0, id1, id2, id3]
    LS = [ls0, ls1]
    IS_ = [is0, is1]
    GS = [gs0, gs1]
    SS = [ss0, ss1]

    zero16 = jnp.zeros((16,), jnp.float32)

    def _zero_eb0():
        @plsc.parallel_loop(0, K, unroll=4)
        def _zrow(i):
            for cc in range(UNITS // 16):
                eb0[i, pl.ds(cc * 16, 16)] = zero16

    _zero_eb0()
    for i in range(DSTRIPE_A // 16):
        zbuf[pl.ds(i * 16, 16)] = zero16
    for i in range(K // 16):
        onesb[pl.ds(i * 16, 16)] = jnp.full((16,), 1.0, jnp.float32)

    row_a = pl.multiple_of(s * STRIPE_A, 16)

    def _zero_acc():
        @pl.when(s < NS - 1)
        def _():
            for i in range(8):
                pltpu.sync_copy(eb0.at[pl.ds(0, STRIPE_A // 8)],
                                acc.at[pl.ds(row_a + i * (STRIPE_A // 8),
                                             STRIPE_A // 8)])

        @pl.when(s == NS - 1)
        def _():
            for i in range(8):
                pltpu.sync_copy(eb0.at[pl.ds(0, STRIPE_LAST // 8)],
                                acc.at[pl.ds(row_a + i * (STRIPE_LAST // 8),
                                             STRIPE_LAST // 8)])

    def _write_acc(out3_h):
        @pl.when(s < NS - 1)
        def _():
            pltpu.sync_copy(acc.at[pl.ds(row_a, STRIPE_A)],
                            out3_h.at[c, pl.ds(row_a, STRIPE_A)])

        @pl.when(s == NS - 1)
        def _():
            pltpu.sync_copy(acc.at[pl.ds(row_a, STRIPE_LAST)],
                            out3_h.at[c, pl.ds(row_a, STRIPE_LAST)])

    _zero_acc()

    drow = pl.multiple_of(s * DSTRIPE_A, 16)

    @pl.when(s < NS - 1)
    def _():
        pltpu.sync_copy(zbuf, dacc.at[pl.ds(drow, DSTRIPE_A)])

    @pl.when(s == NS - 1)
    def _():
        pltpu.sync_copy(zbuf.at[pl.ds(0, DSTRIPE_LAST)],
                        dacc.at[pl.ds((NS - 1) * DSTRIPE_A, DSTRIPE_LAST)])

    plsc.subcore_barrier()

    tile_base = (c * NS + s) * EPT

    def p_issue_load(b, r2, r4):
        ebase = tile_base + b * K
        pltpu.async_copy(e_h.at[pl.ds(ebase, K)], E4[r4], LS[r2])
        pltpu.async_copy(src_h.at[pl.ds(ebase, K)], IXS[r4], IS_[r2])

    def p_wait_load(r2, r4):
        pltpu.make_async_copy(e_h.at[pl.ds(0, K)], E4[r4], LS[r2]).wait()
        pltpu.make_async_copy(src_h.at[pl.ds(0, K)], IXS[r4], IS_[r2]).wait()

    def p_issue_scatter(r2, r4):
        pltpu.async_copy(E4[r4], acc.at[IXS[r4]], SS[r2], add=True)
        pltpu.async_copy(onesb, dacc.at[IXS[r4]], SS[r2], add=True)

    def p_wait_scatter(r2, r4):
        pltpu.make_async_copy(E4[r4], acc.at[IXS[r4]], SS[r2]).wait()
        pltpu.make_async_copy(onesb, dacc.at[IXS[r4]], SS[r2]).wait()

    def p_half(b, bmod, has_prev2, has_next2):
        r2, r4 = bmod % 2, bmod % 4
        if has_prev2:
            p_wait_scatter(r2, (bmod - 2) % 4)
        p_wait_load(r2, r4)
        p_issue_scatter(r2, r4)
        if has_next2:
            p_issue_load(b + 2, r2, (bmod + 2) % 4)

    p_issue_load(0, 0, 0)
    p_issue_load(1, 1, 1)
    p_half(0, 0, False, True)
    p_half(1, 1, False, True)

    def _ploop(g, carry):
        b0 = 2 + 4 * g
        for b2 in range(4):
            p_half(b0 + b2, (2 + b2) % 4, True, True)
        return carry

    lax.fori_loop(0, (NBLK - 5) // 4, _ploop, 0)
    p_half(NBLK - 3, (NBLK - 3) % 4, True, True)
    p_half(NBLK - 2, (NBLK - 2) % 4, True, False)
    p_half(NBLK - 1, (NBLK - 1) % 4, True, False)
    p_wait_scatter((NBLK - 2) % 2, (NBLK - 2) % 4)
    p_wait_scatter((NBLK - 1) % 2, (NBLK - 1) % 4)

    plsc.subcore_barrier()

    _write_acc(p_h)

    @pl.when(s < NS - 1)
    def _():
        pltpu.sync_copy(dacc.at[pl.ds(drow, DSTRIPE_A)], zbuf)

        @pl.when(c == 0)
        def _():
            pltpu.sync_copy(zbuf, deg0_h.at[pl.ds(drow, DSTRIPE_A)])

        @pl.when(c == 1)
        def _():
            pltpu.sync_copy(zbuf, deg1_h.at[pl.ds(drow, DSTRIPE_A)])

    @pl.when(s == NS - 1)
    def _():
        last = pl.ds((NS - 1) * DSTRIPE_A, DSTRIPE_LAST)
        pltpu.sync_copy(dacc.at[last], zbuf.at[pl.ds(0, DSTRIPE_LAST)])

        @pl.when(c == 0)
        def _():
            pltpu.sync_copy(zbuf.at[pl.ds(0, DSTRIPE_LAST)], deg0_h.at[last])

        @pl.when(c == 1)
        def _():
            pltpu.sync_copy(zbuf.at[pl.ds(0, DSTRIPE_LAST)], deg1_h.at[last])

    plsc.subcore_barrier()
    _zero_eb0()
    _zero_acc()
    plsc.subcore_barrier()

    def u_issue_load(b, r2, r4):
        ebase = tile_base + b * K
        pltpu.async_copy(e_h.at[pl.ds(ebase, K)], EU[r2], LS[r2])
        pltpu.async_copy(src_h.at[pl.ds(ebase, K)], IXS[r4], IS_[r2])
        pltpu.async_copy(dst_h.at[pl.ds(ebase, K)], IXD[r4], IS_[r2])

    def u_wait_es(r2):
        pltpu.make_async_copy(e_h.at[pl.ds(0, K)], EU[r2], LS[r2]).wait()

    def u_wait_idx(r2, r4):
        pltpu.make_async_copy(src_h.at[pl.ds(0, K)], IXS[r4], IS_[r2]).wait()
        pltpu.make_async_copy(dst_h.at[pl.ds(0, K)], IXD[r4], IS_[r2]).wait()

    def u_issue_gather(r2, r4):
        pltpu.async_copy(y2_h.at[IXD[r4]], YU[r2], GS[r2])

    def u_wait_gather(r2, r4):
        pltpu.make_async_copy(y2_h.at[IXD[r4]], YU[r2], GS[r2]).wait()

    def u_issue_scatter(r2, r4):
        pltpu.async_copy(YU[r2], acc.at[IXS[r4]], SS[r2], add=True)

    def u_wait_scatter(r2, r4):
        pltpu.make_async_copy(YU[r2], acc.at[IXS[r4]], SS[r2]).wait()

    def u_mul(r2):
        eb, yb = EU[r2], YU[r2]

        @plsc.parallel_loop(0, K, unroll=4)
        def _row(r):
            for cc in range(UNITS // 16):
                sl = pl.ds(cc * 16, 16)
                yb[r, sl] = eb[r, sl] * yb[r, sl]

    def u_half(b, bmod, do_ss_wait, do_load2, do_gather1):
        r2, r4 = bmod % 2, bmod % 4
        o2, o4n = 1 - r2, (bmod + 1) % 4
        u_wait_es(r2)
        u_wait_gather(r2, r4)
        u_mul(r2)
        u_issue_scatter(r2, r4)
        if do_load2:
            u_issue_load(b + 2, r2, (bmod + 2) % 4)
        if do_ss_wait:
            u_wait_scatter(o2, (bmod - 1) % 4)
        if do_gather1:
            u_wait_idx(o2, o4n)
            u_issue_gather(o2, o4n)

    u_issue_load(0, 0, 0)
    u_issue_load(1, 1, 1)
    u_wait_idx(0, 0)
    u_issue_gather(0, 0)
    u_half(0, 0, False, True, True)

    def _uloop(g, carry):
        b0 = 1 + 4 * g
        for b2 in range(4):
            u_half(b0 + b2, (1 + b2) % 4, True, True, True)
        return carry

    lax.fori_loop(0, (NBLK - 5) // 4, _uloop, 0)
    u_half(NBLK - 4, (NBLK - 4) % 4, True, True, True)
    u_half(NBLK - 3, (NBLK - 3) % 4, True, True, True)
    u_half(NBLK - 2, (NBLK - 2) % 4, True, False, True)
    u_half(NBLK - 1, (NBLK - 1) % 4, True, False, False)
    u_wait_scatter((NBLK - 1) % 2, (NBLK - 1) % 4)

    plsc.subcore_barrier()
    _write_acc(u_h)


@functools.cache
def _edge_call_build():
    return pl.kernel(
        _edge_body,
        mesh=plsc.VectorSubcoreMesh(core_axis_name="c", subcore_axis_name="s",
                                    num_cores=NC, num_subcores=NS),
        out_type=[
            jax.ShapeDtypeStruct((NC, N_NODES, UNITS), jnp.float32),
            jax.ShapeDtypeStruct((NC, N_NODES, UNITS), jnp.float32),
            jax.ShapeDtypeStruct((N_NODES,), jnp.float32),
            jax.ShapeDtypeStruct((N_NODES,), jnp.float32),
        ],
        scratch_types=[
            pltpu.VMEM((K, UNITS), jnp.float32),
            pltpu.VMEM((K, UNITS), jnp.float32),
            pltpu.VMEM((K, UNITS), jnp.float32),
            pltpu.VMEM((K, UNITS), jnp.float32),
            pltpu.VMEM((K,), jnp.int32),
            pltpu.VMEM((K,), jnp.int32),
            pltpu.VMEM((K,), jnp.int32),
            pltpu.VMEM((K,), jnp.int32),
            pltpu.VMEM((K,), jnp.int32),
            pltpu.VMEM((K,), jnp.int32),
            pltpu.VMEM((K,), jnp.int32),
            pltpu.VMEM((K,), jnp.int32),
            pltpu.VMEM((K,), jnp.float32),
            pltpu.VMEM((DSTRIPE_A,), jnp.float32),
            pltpu.VMEM_SHARED((N_NODES, UNITS), jnp.float32),
            pltpu.VMEM_SHARED((N_NODES,), jnp.float32),
            pltpu.SemaphoreType.DMA,
            pltpu.SemaphoreType.DMA,
            pltpu.SemaphoreType.DMA,
            pltpu.SemaphoreType.DMA,
            pltpu.SemaphoreType.DMA,
            pltpu.SemaphoreType.DMA,
            pltpu.SemaphoreType.DMA,
            pltpu.SemaphoreType.DMA,
        ],
    )


def _final_body(x_ref, w1_ref, p_ref, u_ref, d0_ref, d1_ref, g_ref, b_ref,
                o_ref):
    x = x_ref[...]
    y1 = jnp.dot(x, w1_ref[...].T, preferred_element_type=jnp.float32)
    p = p_ref[0] + p_ref[1]
    u = u_ref[0] + u_ref[1]
    deg = d0_ref[...] + d1_ref[...]
    h = deg * y1 + u / (p + EPS)
    mean = jnp.mean(h, axis=0, keepdims=True)
    ctr = h - mean
    var = jnp.mean(ctr * ctr, axis=0, keepdims=True)
    hn = ctr * lax.rsqrt(var + BN_EPS) * g_ref[...] + b_ref[...]
    o_ref[...] = x + jnp.maximum(hn, 0.0)


def _final_call(x, W1, p2, u2, deg0, deg1, gamma, beta):
    return pl.pallas_call(
        _final_body,
        out_shape=jax.ShapeDtypeStruct((N_NODES, UNITS), jnp.float32),
    )(x, W1, p2, u2, deg0, deg1, gamma, beta)


def kernel(x, e, edge_index, W1, W2, gamma, beta):
    src = edge_index[0].astype(jnp.int32)
    dst = edge_index[1].astype(jnp.int32)
    y2 = _y2_call(x, W2)
    es = _es_call(e)
    p2, u2, deg0, deg1 = _edge_call_build()(es, src, dst, y2)
    return _final_call(x, W1, p2, u2,
                       deg0.reshape(N_NODES, 1), deg1.reshape(N_NODES, 1),
                       gamma.reshape(1, UNITS), beta.reshape(1, UNITS))


# speedup vs baseline: 1.0658x; 1.0658x over previous
import functools

import jax
import jax.numpy as jnp
from jax import lax
from jax.experimental import pallas as pl
from jax.experimental.pallas import tpu as pltpu
from jax.experimental.pallas import tpu_sc as plsc

N_NODES = 10000
N_EDGES = 320000
UNITS = 128
EPS = 1e-08
BN_EPS = 1e-05

NC = 2
NS = 16

EPT = N_EDGES // (NC * NS)
K = 80
NBLK = EPT // K

STRIPE_A = 624
STRIPE_LAST = N_NODES - (NS - 1) * STRIPE_A
DSTRIPE_A = 640
DSTRIPE_LAST = N_NODES - (NS - 1) * DSTRIPE_A


def _y2_body(x_ref, w2_ref, y_ref):
    y_ref[...] = jnp.dot(x_ref[...], w2_ref[...].T,
                         preferred_element_type=jnp.float32)


def _y2_call(x, W2):
    return pl.pallas_call(
        _y2_body,
        out_shape=jax.ShapeDtypeStruct((N_NODES, UNITS), jnp.float32),
    )(x, W2)


def _edge_body(e_h, srd_h, y2_h,
               p_h, u_h, deg0_h, deg1_h,
               eb0, eb1, yb0, yb1,
               ix0, ix1, ix2, ix3,
               onesb, zbuf,
               acc, dacc,
               ls0, ls1, is0, is1, gs0, gs1, ss0, ss1):
    c = lax.axis_index("c")
    s = lax.axis_index("s")

    E4 = [eb0, eb1, yb0, yb1]
    EU = [eb0, eb1]
    YU = [yb0, yb1]
    IX = [ix0, ix1, ix2, ix3]
    LS = [ls0, ls1]
    IS_ = [is0, is1]
    GS = [gs0, gs1]
    SS = [ss0, ss1]

    zero16 = jnp.zeros((16,), jnp.float32)

    def _zero_eb0():
        @plsc.parallel_loop(0, K, unroll=4)
        def _zrow(i):
            for cc in range(UNITS // 16):
                eb0[i, pl.ds(cc * 16, 16)] = zero16

    _zero_eb0()
    for i in range(DSTRIPE_A // 16):
        zbuf[pl.ds(i * 16, 16)] = zero16
    for i in range(K // 16):
        onesb[pl.ds(i * 16, 16)] = jnp.full((16,), 1.0, jnp.float32)

    row_a = pl.multiple_of(s * STRIPE_A, 16)

    def _zero_acc():
        @pl.when(s < NS - 1)
        def _():
            for i in range(8):
                pltpu.sync_copy(eb0.at[pl.ds(0, STRIPE_A // 8)],
                                acc.at[pl.ds(row_a + i * (STRIPE_A // 8),
                                             STRIPE_A // 8)])

        @pl.when(s == NS - 1)
        def _():
            for i in range(8):
                pltpu.sync_copy(eb0.at[pl.ds(0, STRIPE_LAST // 8)],
                                acc.at[pl.ds(row_a + i * (STRIPE_LAST // 8),
                                             STRIPE_LAST // 8)])

    def _write_acc(out3_h):
        @pl.when(s < NS - 1)
        def _():
            pltpu.sync_copy(acc.at[pl.ds(row_a, STRIPE_A)],
                            out3_h.at[c, pl.ds(row_a, STRIPE_A)])

        @pl.when(s == NS - 1)
        def _():
            pltpu.sync_copy(acc.at[pl.ds(row_a, STRIPE_LAST)],
                            out3_h.at[c, pl.ds(row_a, STRIPE_LAST)])

    _zero_acc()

    drow = pl.multiple_of(s * DSTRIPE_A, 16)

    @pl.when(s < NS - 1)
    def _():
        pltpu.sync_copy(zbuf, dacc.at[pl.ds(drow, DSTRIPE_A)])

    @pl.when(s == NS - 1)
    def _():
        pltpu.sync_copy(zbuf.at[pl.ds(0, DSTRIPE_LAST)],
                        dacc.at[pl.ds((NS - 1) * DSTRIPE_A, DSTRIPE_LAST)])

    plsc.subcore_barrier()

    tile_base = (c * NS + s) * EPT

    def _block_row(b):
        return (c * NS + s) * NBLK + b

    def p_issue_load(b, r2, r4):
        ebase = tile_base + b * K
        pltpu.async_copy(e_h.at[pl.ds(ebase, K)], E4[r4], LS[r2])
        pltpu.async_copy(srd_h.at[_block_row(b)], IX[r4], IS_[r2])

    def p_wait_load(r2, r4):
        pltpu.make_async_copy(e_h.at[pl.ds(0, K)], E4[r4], LS[r2]).wait()
        pltpu.make_async_copy(srd_h.at[0], IX[r4], IS_[r2]).wait()

    def p_sigmoid(r4):
        eb = E4[r4]

        @plsc.parallel_loop(0, K, unroll=2)
        def _row(r):
            for cc in range(UNITS // 16):
                sl = pl.ds(cc * 16, 16)
                v = eb[r, sl]
                eb[r, sl] = 1.0 / (1.0 + jnp.exp(-v))

    def p_issue_scatter(r2, r4):
        pltpu.async_copy(E4[r4], acc.at[IX[r4].at[0]], SS[r2], add=True)
        pltpu.async_copy(onesb, dacc.at[IX[r4].at[0]], SS[r2], add=True)

    def p_wait_scatter(r2, r4):
        pltpu.make_async_copy(E4[r4], acc.at[IX[r4].at[0]], SS[r2]).wait()
        pltpu.make_async_copy(onesb, dacc.at[IX[r4].at[0]], SS[r2]).wait()

    def p_half(b, bmod, has_prev2, has_next2):
        r2, r4 = bmod % 2, bmod % 4
        if has_prev2:
            p_wait_scatter(r2, (bmod - 2) % 4)
        p_wait_load(r2, r4)
        p_sigmoid(r4)
        p_issue_scatter(r2, r4)
        if has_next2:
            p_issue_load(b + 2, r2, (bmod + 2) % 4)

    p_issue_load(0, 0, 0)
    p_issue_load(1, 1, 1)
    p_half(0, 0, False, True)
    p_half(1, 1, False, True)

    def _ploop(g, carry):
        b0 = 2 + 4 * g
        for b2 in range(4):
            p_half(b0 + b2, (2 + b2) % 4, True, True)
        return carry

    lax.fori_loop(0, (NBLK - 5) // 4, _ploop, 0)
    p_half(NBLK - 3, (NBLK - 3) % 4, True, True)
    p_half(NBLK - 2, (NBLK - 2) % 4, True, False)
    p_half(NBLK - 1, (NBLK - 1) % 4, True, False)
    p_wait_scatter((NBLK - 2) % 2, (NBLK - 2) % 4)
    p_wait_scatter((NBLK - 1) % 2, (NBLK - 1) % 4)

    plsc.subcore_barrier()

    _write_acc(p_h)

    @pl.when(s < NS - 1)
    def _():
        pltpu.sync_copy(dacc.at[pl.ds(drow, DSTRIPE_A)], zbuf)

        @pl.when(c == 0)
        def _():
            pltpu.sync_copy(zbuf, deg0_h.at[pl.ds(drow, DSTRIPE_A)])

        @pl.when(c == 1)
        def _():
            pltpu.sync_copy(zbuf, deg1_h.at[pl.ds(drow, DSTRIPE_A)])

    @pl.when(s == NS - 1)
    def _():
        last = pl.ds((NS - 1) * DSTRIPE_A, DSTRIPE_LAST)
        pltpu.sync_copy(dacc.at[last], zbuf.at[pl.ds(0, DSTRIPE_LAST)])

        @pl.when(c == 0)
        def _():
            pltpu.sync_copy(zbuf.at[pl.ds(0, DSTRIPE_LAST)], deg0_h.at[last])

        @pl.when(c == 1)
        def _():
            pltpu.sync_copy(zbuf.at[pl.ds(0, DSTRIPE_LAST)], deg1_h.at[last])

    plsc.subcore_barrier()
    _zero_eb0()
    _zero_acc()
    plsc.subcore_barrier()

    def u_issue_load(b, r2, r4):
        ebase = tile_base + b * K
        pltpu.async_copy(e_h.at[pl.ds(ebase, K)], EU[r2], LS[r2])
        pltpu.async_copy(srd_h.at[_block_row(b)], IX[r4], IS_[r2])

    def u_wait_es(r2):
        pltpu.make_async_copy(e_h.at[pl.ds(0, K)], EU[r2], LS[r2]).wait()

    def u_wait_idx(r2, r4):
        pltpu.make_async_copy(srd_h.at[0], IX[r4], IS_[r2]).wait()

    def u_issue_gather(r2, r4):
        pltpu.async_copy(y2_h.at[IX[r4].at[1]], YU[r2], GS[r2])

    def u_wait_gather(r2, r4):
        pltpu.make_async_copy(y2_h.at[IX[r4].at[1]], YU[r2], GS[r2]).wait()

    def u_issue_scatter(r2, r4):
        pltpu.async_copy(YU[r2], acc.at[IX[r4].at[0]], SS[r2], add=True)

    def u_wait_scatter(r2, r4):
        pltpu.make_async_copy(YU[r2], acc.at[IX[r4].at[0]], SS[r2]).wait()

    def u_mul(r2):
        eb, yb = EU[r2], YU[r2]

        @plsc.parallel_loop(0, K, unroll=2)
        def _row(r):
            for cc in range(UNITS // 16):
                sl = pl.ds(cc * 16, 16)
                v = eb[r, sl]
                sg = 1.0 / (1.0 + jnp.exp(-v))
                yb[r, sl] = sg * yb[r, sl]

    def u_half(b, bmod, do_ss_wait, do_load2, do_gather1):
        r2, r4 = bmod % 2, bmod % 4
        o2, o4n = 1 - r2, (bmod + 1) % 4
        u_wait_es(r2)
        u_wait_gather(r2, r4)
        u_mul(r2)
        u_issue_scatter(r2, r4)
        if do_load2:
            u_issue_load(b + 2, r2, (bmod + 2) % 4)
        if do_ss_wait:
            u_wait_scatter(o2, (bmod - 1) % 4)
        if do_gather1:
            u_wait_idx(o2, o4n)
            u_issue_gather(o2, o4n)

    u_issue_load(0, 0, 0)
    u_issue_load(1, 1, 1)
    u_wait_idx(0, 0)
    u_issue_gather(0, 0)
    u_half(0, 0, False, True, True)

    def _uloop(g, carry):
        b0 = 1 + 4 * g
        for b2 in range(4):
            u_half(b0 + b2, (1 + b2) % 4, True, True, True)
        return carry

    lax.fori_loop(0, (NBLK - 5) // 4, _uloop, 0)
    u_half(NBLK - 4, (NBLK - 4) % 4, True, True, True)
    u_half(NBLK - 3, (NBLK - 3) % 4, True, True, True)
    u_half(NBLK - 2, (NBLK - 2) % 4, True, False, True)
    u_half(NBLK - 1, (NBLK - 1) % 4, True, False, False)
    u_wait_scatter((NBLK - 1) % 2, (NBLK - 1) % 4)

    plsc.subcore_barrier()
    _write_acc(u_h)


@functools.cache
def _edge_call_build():
    return pl.kernel(
        _edge_body,
        mesh=plsc.VectorSubcoreMesh(core_axis_name="c", subcore_axis_name="s",
                                    num_cores=NC, num_subcores=NS),
        out_type=[
            jax.ShapeDtypeStruct((NC, N_NODES, UNITS), jnp.float32),
            jax.ShapeDtypeStruct((NC, N_NODES, UNITS), jnp.float32),
            jax.ShapeDtypeStruct((N_NODES,), jnp.float32),
            jax.ShapeDtypeStruct((N_NODES,), jnp.float32),
        ],
        scratch_types=[
            pltpu.VMEM((K, UNITS), jnp.float32),
            pltpu.VMEM((K, UNITS), jnp.float32),
            pltpu.VMEM((K, UNITS), jnp.float32),
            pltpu.VMEM((K, UNITS), jnp.float32),
            pltpu.VMEM((2, K), jnp.int32),
            pltpu.VMEM((2, K), jnp.int32),
            pltpu.VMEM((2, K), jnp.int32),
            pltpu.VMEM((2, K), jnp.int32),
            pltpu.VMEM((K,), jnp.float32),
            pltpu.VMEM((DSTRIPE_A,), jnp.float32),
            pltpu.VMEM_SHARED((N_NODES, UNITS), jnp.float32),
            pltpu.VMEM_SHARED((N_NODES,), jnp.float32),
            pltpu.SemaphoreType.DMA,
            pltpu.SemaphoreType.DMA,
            pltpu.SemaphoreType.DMA,
            pltpu.SemaphoreType.DMA,
            pltpu.SemaphoreType.DMA,
            pltpu.SemaphoreType.DMA,
            pltpu.SemaphoreType.DMA,
            pltpu.SemaphoreType.DMA,
        ],
    )


def _final_body(x_ref, w1_ref, p_ref, u_ref, d0_ref, d1_ref, g_ref, b_ref,
                o_ref):
    x = x_ref[...]
    y1 = jnp.dot(x, w1_ref[...].T, preferred_element_type=jnp.float32)
    p = p_ref[0] + p_ref[1]
    u = u_ref[0] + u_ref[1]
    deg = d0_ref[...] + d1_ref[...]
    h = deg * y1 + u / (p + EPS)
    mean = jnp.mean(h, axis=0, keepdims=True)
    ctr = h - mean
    var = jnp.mean(ctr * ctr, axis=0, keepdims=True)
    hn = ctr * lax.rsqrt(var + BN_EPS) * g_ref[...] + b_ref[...]
    o_ref[...] = x + jnp.maximum(hn, 0.0)


def _final_call(x, W1, p2, u2, deg0, deg1, gamma, beta):
    return pl.pallas_call(
        _final_body,
        out_shape=jax.ShapeDtypeStruct((N_NODES, UNITS), jnp.float32),
    )(x, W1, p2, u2, deg0, deg1, gamma, beta)


def kernel(x, e, edge_index, W1, W2, gamma, beta):
    srd = edge_index.astype(jnp.int32).reshape(2, NC * NS * NBLK, K)
    srd = srd.transpose(1, 0, 2)
    y2 = _y2_call(x, W2)
    p2, u2, deg0, deg1 = _edge_call_build()(e, srd, y2)
    return _final_call(x, W1, p2, u2,
                       deg0.reshape(N_NODES, 1), deg1.reshape(N_NODES, 1),
                       gamma.reshape(1, UNITS), beta.reshape(1, UNITS))
